# baseline (speedup 1.0000x reference)
import jax
import jax.numpy as jnp
from jax import lax
from jax.experimental import pallas as pl
from jax.experimental.pallas import tpu as pltpu
from jax.experimental.pallas import tpu_sc as plsc

MAXLEN = 200
EMB = 32
BATCH = 4096
VOCAB = 2 * MAXLEN + 2
PAD_MAX = 2 * MAXLEN + 1

CHUNK_ROWS = 8
CHUNK_OUT = CHUNK_ROWS * MAXLEN
LANES = 16


def _build_indices(idx_ref, start_vec, len_vec, lane, r0):
    for r in range(CHUNK_ROWS):
        start_s = start_vec[r0 + r]
        len_s = len_vec[r0 + r]
        for g in range(13):
            off = 16 * g if g < 12 else MAXLEN - LANES
            j_vec = lane + off
            rel = jnp.clip(j_vec + start_s, 1, PAD_MAX)
            idx = jnp.where(j_vec < len_s, rel, 0)
            idx_ref[pl.ds(r * MAXLEN + off, LANES)] = idx


def _fire_outs(rows_ref, out_hbm, row_base, osem):
    copies = []
    for r in range(CHUNK_ROWS):
        copies.append(
            pltpu.async_copy(
                rows_ref.at[pl.ds(r * MAXLEN, MAXLEN)],
                out_hbm.at[row_base + r, :, pl.ds(0, EMB)],
                osem,
            )
        )
    return copies


def _drain_outs(rows_ref, out_hbm, row_base, osem):
    for r in range(CHUNK_ROWS):
        pltpu.make_async_copy(
            rows_ref.at[pl.ds(r * MAXLEN, MAXLEN)],
            out_hbm.at[row_base + r, :, pl.ds(0, EMB)],
            osem,
        ).wait()


def _body(pos_hbm, len_hbm, table_hbm, out_hbm, tab_s, pos_v, len_v,
          idx_a, idx_b, rows_a, rows_b, gsem_a, gsem_b, osem_a, osem_b):
    info = plsc.get_sparse_core_info()
    nc = info.num_cores
    nw = nc * info.num_subcores
    rows_per_worker = BATCH // nw
    num_pairs = rows_per_worker // (2 * CHUNK_ROWS)

    sid = lax.axis_index("s")
    wid = sid * nc + lax.axis_index("c")
    base = wid * rows_per_worker

    @pl.when(sid == 0)
    def _():
        pltpu.sync_copy(table_hbm, tab_s)

    pltpu.sync_copy(pos_hbm.at[pl.ds(base, rows_per_worker)], pos_v)
    pltpu.sync_copy(len_hbm.at[pl.ds(base, rows_per_worker)], len_v)
    plsc.subcore_barrier()
    lane = lax.iota(jnp.int32, LANES)

    def pair_body(k, carry):
        pos_vec = pos_v[pl.ds(k * 2 * CHUNK_ROWS, LANES)]
        len_vec = len_v[pl.ds(k * 2 * CHUNK_ROWS, LANES)]
        start_vec = (MAXLEN + 1) - pos_vec
        row_a = base + k * 2 * CHUNK_ROWS
        row_b = row_a + CHUNK_ROWS

        _build_indices(idx_a, start_vec, len_vec, lane, 0)

        @pl.when(k > 0)
        def _():
            _drain_outs(rows_a, out_hbm, row_a, osem_a)

        ga = pltpu.async_copy(tab_s.at[idx_a], rows_a, gsem_a)

        _build_indices(idx_b, start_vec, len_vec, lane, CHUNK_ROWS)

        @pl.when(k > 0)
        def _():
            _drain_outs(rows_b, out_hbm, row_b, osem_b)

        gb = pltpu.async_copy(tab_s.at[idx_b], rows_b, gsem_b)

        ga.wait()
        _fire_outs(rows_a, out_hbm, row_a, osem_a)
        gb.wait()
        _fire_outs(rows_b, out_hbm, row_b, osem_b)
        return carry

    lax.fori_loop(0, num_pairs, pair_body, 0)
    last_a = base + rows_per_worker - 2 * CHUNK_ROWS
    _drain_outs(rows_a, out_hbm, last_a, osem_a)
    _drain_outs(rows_b, out_hbm, last_a + CHUNK_ROWS, osem_b)


def kernel(positions, lengths, table):
    info = plsc.get_sparse_core_info()
    nw = info.num_cores * info.num_subcores
    rows_per_worker = BATCH // nw
    mesh = plsc.VectorSubcoreMesh(core_axis_name="c", subcore_axis_name="s")
    k = pl.kernel(
        _body,
        out_type=jax.ShapeDtypeStruct((BATCH, MAXLEN, 128), jnp.float32),
        mesh=mesh,
        compiler_params=pltpu.CompilerParams(use_tc_tiling_on_sc=False),
        scratch_types=[
            pltpu.VMEM_SHARED((VOCAB, EMB), jnp.float32),
            pltpu.VMEM((rows_per_worker,), jnp.int32),
            pltpu.VMEM((rows_per_worker,), jnp.int32),
            pltpu.VMEM((CHUNK_OUT,), jnp.int32),
            pltpu.VMEM((CHUNK_OUT,), jnp.int32),
            pltpu.VMEM((CHUNK_OUT, EMB), jnp.float32),
            pltpu.VMEM((CHUNK_OUT, EMB), jnp.float32),
            pltpu.SemaphoreType.DMA,
            pltpu.SemaphoreType.DMA,
            pltpu.SemaphoreType.DMA,
            pltpu.SemaphoreType.DMA,
        ],
    )
    padded = k(positions.astype(jnp.int32), lengths.astype(jnp.int32), table)
    return padded[:, :, :EMB]
